# baseline (speedup 1.0000x reference)
import functools

import jax
import jax.numpy as jnp
from jax import lax
from jax.experimental import pallas as pl
from jax.experimental.pallas import tpu as pltpu
from jax.experimental.pallas import tpu_sc as plsc

_B = 16384
_D = 1000


_K = 8
_G = 4
_TB = _B // (_G * _K)


def _dense_body(*refs):
    w_ref = refs[0]
    x_refs = refs[1:-1]
    out_ref = refs[-1]
    i = pl.program_id(0)

    @pl.when(i == 0)
    def _init():
        out_ref[...] = jnp.zeros((1, 1), jnp.float32)

    w = w_ref[...]
    part = jnp.float32(0.0)
    for x_ref in x_refs:
        x = x_ref[...]
        part = part + jnp.sum(w * x * x)
    out_ref[...] = out_ref[...] + part


def _dense_sum(x, w):
    def _imap(j):
        return lambda i: (i * _K + j, 0)

    return pl.pallas_call(
        _dense_body,
        grid=(_G,),
        in_specs=[pl.BlockSpec((1, _D), lambda i: (0, 0))]
        + [pl.BlockSpec((_TB, _D), _imap(j)) for j in range(_K)],
        out_specs=pl.BlockSpec((1, 1), lambda i: (0, 0)),
        out_shape=jax.ShapeDtypeStruct((1, 1), jnp.float32),
    )(w.reshape(1, _D), *([x] * _K))


_NC = 2
_NS = 16
_NW = _NC * _NS
_BPW = _B // _NW
_NCHUNK = _BPW // 16
_NIDX = _BPW // 128


@functools.partial(
    pl.kernel,
    mesh=plsc.VectorSubcoreMesh(core_axis_name="c", subcore_axis_name="s"),
    out_type=jax.ShapeDtypeStruct((_NW, 16), jnp.float32),
    scratch_types=[
        pltpu.VMEM((_BPW,), jnp.int32),
        pltpu.VMEM((_NIDX, 128), jnp.int32),
        pltpu.VMEM((_NIDX, 128), jnp.int32),
        pltpu.VMEM((_NIDX, 128), jnp.float32),
        pltpu.VMEM((_NIDX, 128), jnp.float32),
        pltpu.VMEM((16,), jnp.float32),
        pltpu.SemaphoreType.DMA,
    ],
)
def _corr_kernel(xflat_hbm, tgt_hbm, w_hbm, out_hbm,
                 tgt_v, idx_v, tdx_v, xs_v, ws_v, o_v, sem):
    wid = lax.axis_index("s") * _NC + lax.axis_index("c")
    base = wid * _BPW
    pltpu.sync_copy(tgt_hbm.at[pl.ds(base, _BPW)], tgt_v)

    for i in range(_NCHUNK):
        t16 = tgt_v[pl.ds(i * 16, 16)]
        rows = base + i * 16 + lax.broadcasted_iota(jnp.int32, (16,), 0)
        idx_v[i // 8, pl.ds((i % 8) * 16, 16)] = rows * _D + t16
        tdx_v[i // 8, pl.ds((i % 8) * 16, 16)] = t16

    copies = [
        pltpu.async_copy(xflat_hbm.at[idx_v.at[j]], xs_v.at[j], sem)
        for j in range(_NIDX)
    ] + [
        pltpu.async_copy(w_hbm.at[tdx_v.at[j]], ws_v.at[j], sem)
        for j in range(_NIDX)
    ]
    for cp in copies:
        cp.wait()

    acc = jnp.zeros((16,), jnp.float32)
    for i in range(_NCHUNK):
        x16 = xs_v[i // 8, pl.ds((i % 8) * 16, 16)]
        w16 = ws_v[i // 8, pl.ds((i % 8) * 16, 16)]
        acc = acc + w16 * (1.0 - 2.0 * x16)
    o_v[...] = acc
    pltpu.sync_copy(o_v, out_hbm.at[wid])


def kernel(inputs, targets, loss_weights):
    dense = _dense_sum(inputs, loss_weights)
    xt = inputs[jnp.arange(_B), targets]
    corr = jnp.sum(loss_weights[targets] * (1.0 - 2.0 * xt))
    total = dense[0, 0] + corr
    return total / jnp.float32(_B * _D)


# speedup vs baseline: 1.2523x; 1.2523x over previous
import functools

import jax
import jax.numpy as jnp
from jax import lax
from jax.experimental import pallas as pl
from jax.experimental.pallas import tpu as pltpu
from jax.experimental.pallas import tpu_sc as plsc

_B = 16384
_D = 1000


_DW = _D * 16
_ROWS = _B * _D // _DW
_G = 16
_TB = _ROWS // _G


def _dense_body(w_ref, x_ref, out_ref):
    i = pl.program_id(0)

    @pl.when(i == 0)
    def _init():
        out_ref[...] = jnp.zeros((1, 1), jnp.float32)

    x = x_ref[...]
    out_ref[...] = out_ref[...] + jnp.sum(w_ref[...] * x * x)


def _dense_sum(x, w):
    return pl.pallas_call(
        _dense_body,
        grid=(_G,),
        in_specs=[
            pl.BlockSpec((1, _DW), lambda i: (0, 0)),
            pl.BlockSpec((_TB, _DW), lambda i: (i, 0)),
        ],
        out_specs=pl.BlockSpec((1, 1), lambda i: (0, 0)),
        out_shape=jax.ShapeDtypeStruct((1, 1), jnp.float32),
    )(jnp.tile(w, 16).reshape(1, _DW), x.reshape(_ROWS, _DW))


_NC = 2
_NS = 16
_NW = _NC * _NS
_BPW = _B // _NW
_NCHUNK = _BPW // 16
_NIDX = _BPW // 128


@functools.partial(
    pl.kernel,
    mesh=plsc.VectorSubcoreMesh(core_axis_name="c", subcore_axis_name="s"),
    out_type=jax.ShapeDtypeStruct((_NW, 16), jnp.float32),
    scratch_types=[
        pltpu.VMEM((_BPW,), jnp.int32),
        pltpu.VMEM((_NIDX, 128), jnp.int32),
        pltpu.VMEM((_NIDX, 128), jnp.int32),
        pltpu.VMEM((_NIDX, 128), jnp.float32),
        pltpu.VMEM((_NIDX, 128), jnp.float32),
        pltpu.VMEM((16,), jnp.float32),
        pltpu.SemaphoreType.DMA,
    ],
)
def _corr_kernel(xflat_hbm, tgt_hbm, w_hbm, out_hbm,
                 tgt_v, idx_v, tdx_v, xs_v, ws_v, o_v, sem):
    wid = lax.axis_index("s") * _NC + lax.axis_index("c")
    base = wid * _BPW
    pltpu.sync_copy(tgt_hbm.at[pl.ds(base, _BPW)], tgt_v)

    for i in range(_NCHUNK):
        t16 = tgt_v[pl.ds(i * 16, 16)]
        rows = base + i * 16 + lax.broadcasted_iota(jnp.int32, (16,), 0)
        idx_v[i // 8, pl.ds((i % 8) * 16, 16)] = rows * _D + t16
        tdx_v[i // 8, pl.ds((i % 8) * 16, 16)] = t16

    copies = [
        pltpu.async_copy(xflat_hbm.at[idx_v.at[j]], xs_v.at[j], sem)
        for j in range(_NIDX)
    ] + [
        pltpu.async_copy(w_hbm.at[tdx_v.at[j]], ws_v.at[j], sem)
        for j in range(_NIDX)
    ]
    for cp in copies:
        cp.wait()

    acc = jnp.zeros((16,), jnp.float32)
    for i in range(_NCHUNK):
        x16 = xs_v[i // 8, pl.ds((i % 8) * 16, 16)]
        w16 = ws_v[i // 8, pl.ds((i % 8) * 16, 16)]
        acc = acc + w16 * (1.0 - 2.0 * x16)
    o_v[...] = acc
    pltpu.sync_copy(o_v, out_hbm.at[wid])


def kernel(inputs, targets, loss_weights):
    dense = _dense_sum(inputs, loss_weights)
    xt = inputs[jnp.arange(_B), targets]
    corr = jnp.sum(loss_weights[targets] * (1.0 - 2.0 * xt))
    total = dense[0, 0] + corr
    return total / jnp.float32(_B * _D)


# speedup vs baseline: 1.3908x; 1.1106x over previous
import functools

import jax
import jax.numpy as jnp
from jax import lax
from jax.experimental import pallas as pl
from jax.experimental.pallas import tpu as pltpu
from jax.experimental.pallas import tpu_sc as plsc

_B = 16384
_D = 1000

_NC = 2
_NS = 16
_NW = _NC * _NS
_BPW = _B // _NW
_FPW = _BPW * _D
_SR = 2 * _D
_CH = 16 * _SR
_NCHK = _FPW // _CH
_NACC = 8
_NCHUNK = _BPW // 16
_NIDX = _BPW // 128


_UR = 4


@functools.partial(
    pl.kernel,
    mesh=plsc.VectorSubcoreMesh(core_axis_name="c", subcore_axis_name="s"),
    out_type=jax.ShapeDtypeStruct((_NW, 16), jnp.float32),
    scratch_types=[
        pltpu.VMEM((2 * _CH,), jnp.float32),
        pltpu.VMEM((_SR,), jnp.float32),
        pltpu.VMEM((_BPW,), jnp.int32),
        pltpu.VMEM((_NIDX, 128), jnp.int32),
        pltpu.VMEM((_NIDX, 128), jnp.int32),
        pltpu.VMEM((_NIDX, 128), jnp.float32),
        pltpu.VMEM((_NIDX, 128), jnp.float32),
        pltpu.VMEM((16,), jnp.float32),
        pltpu.SemaphoreType.DMA,
        pltpu.SemaphoreType.DMA,
    ],
)
def _loss_kernel(xflat_hbm, tgt_hbm, w_hbm, w2_hbm, out_hbm,
                 xb_v, w2_v, tgt_v, idx_v, tdx_v, xs_v, ws_v, o_v,
                 gsem, ssem):
    wid = lax.axis_index("s") * _NC + lax.axis_index("c")
    base = wid * _FPW

    pltpu.sync_copy(tgt_hbm.at[pl.ds(wid * _BPW, _BPW)], tgt_v)
    for i in range(_NCHUNK):
        t16 = tgt_v[pl.ds(i * 16, 16)]
        rows = wid * _BPW + i * 16 + lax.broadcasted_iota(jnp.int32, (16,), 0)
        idx_v[i // 8, pl.ds((i % 8) * 16, 16)] = rows * _D + t16
        tdx_v[i // 8, pl.ds((i % 8) * 16, 16)] = t16
    gcopies = [
        pltpu.async_copy(xflat_hbm.at[idx_v.at[j]], xs_v.at[j], gsem)
        for j in range(_NIDX)
    ] + [
        pltpu.async_copy(w_hbm.at[tdx_v.at[j]], ws_v.at[j], gsem)
        for j in range(_NIDX)
    ]

    pltpu.sync_copy(w2_hbm, w2_v)
    pltpu.async_copy(
        xflat_hbm.at[pl.ds(base, _CH)], xb_v.at[pl.ds(0, _CH)], ssem)

    def _chunk_body(k, accs):
        half = lax.rem(k, 2)

        @pl.when(k + 1 < _NCHK)
        def _start_next():
            pltpu.async_copy(
                xflat_hbm.at[pl.ds(base + (k + 1) * _CH, _CH)],
                xb_v.at[pl.ds(lax.rem(k + 1, 2) * _CH, _CH)], ssem)

        pltpu.make_async_copy(
            xflat_hbm.at[pl.ds(0, _CH)], xb_v.at[pl.ds(0, _CH)], ssem).wait()

        def _group_body(g, accs):
            accs = list(accs)
            off0 = half * _CH + g * (_UR * _SR)
            n = 0
            for c2 in range(_SR // 16):
                wv = w2_v[pl.ds(c2 * 16, 16)]
                for u in range(_UR):
                    xv = xb_v[pl.ds(off0 + u * _SR + c2 * 16, 16)]
                    accs[n % _NACC] = accs[n % _NACC] + wv * (xv * xv)
                    n += 1
            return tuple(accs)

        return lax.fori_loop(0, _CH // (_UR * _SR), _group_body, accs)

    accs = lax.fori_loop(
        0, _NCHK, _chunk_body,
        tuple(jnp.zeros((16,), jnp.float32) for _ in range(_NACC)))

    for cp in gcopies:
        cp.wait()
    acc = accs[0]
    for a in accs[1:]:
        acc = acc + a
    for i in range(_NCHUNK):
        x16 = xs_v[i // 8, pl.ds((i % 8) * 16, 16)]
        w16 = ws_v[i // 8, pl.ds((i % 8) * 16, 16)]
        acc = acc + w16 * (1.0 - 2.0 * x16)
    o_v[...] = acc
    pltpu.sync_copy(o_v, out_hbm.at[wid])


def kernel(inputs, targets, loss_weights):
    xflat = inputs.reshape(_B * _D)
    w2 = jnp.concatenate([loss_weights, loss_weights])
    parts = _loss_kernel(xflat, targets, loss_weights, w2)
    return jnp.sum(parts) / jnp.float32(_B * _D)
